# Initial kernel scaffold; baseline (speedup 1.0000x reference)
import jax
import jax.numpy as jnp
from jax.experimental import pallas as pl


def kernel(proposals, labels):
    raise NotImplementedError("write your pallas kernel here")


# speedup vs baseline: 1380.5661x; 1380.5661x over previous
import functools

import jax
import jax.numpy as jnp
from jax import lax
from jax.experimental import pallas as pl
from jax.experimental.pallas import tpu as pltpu
from jax.experimental.pallas import tpu_sc as plsc

FPS = 25.0
N = 20000
M = 128
NPAD = 20480
ROWS = NPAD // 128
NG = NPAD // 16
BLK = 32
GRID1 = ROWS // BLK


def _iou_pack_kernel(ps_ref, pe_ref, dp_ref, ls_ref, le_ref, dl_ref,
                     w0_ref, w1_ref, w2_ref, w3_ref):
    ps = ps_ref[...]
    pe = pe_ref[...]
    dp = dp_ref[...]
    accs = [jnp.zeros((BLK, 128), jnp.int32) for _ in range(4)]
    for j in range(M):
        ls = ls_ref[j]
        le = le_ref[j]
        dl = dl_ref[j]
        imin = jnp.maximum(ps, ls)
        imax = jnp.minimum(pe, le)
        inter = jnp.maximum(imax - imin, jnp.float32(0.0))
        union = dp + dl - inter
        tp = ((inter / union) > jnp.float32(0.5)).astype(jnp.int32)
        w = j // 32
        b = j % 32
        accs[w] = accs[w] | (tp << b)
    w0_ref[...] = accs[0]
    w1_ref[...] = accs[1]
    w2_ref[...] = accs[2]
    w3_ref[...] = accs[3]


def _stage1(ps2, pe2, dp2, ls, le, dl):
    blk = pl.BlockSpec((BLK, 128), lambda r: (r, 0))
    smem = pl.BlockSpec(memory_space=pltpu.SMEM)
    out = jax.ShapeDtypeStruct((ROWS, 128), jnp.int32)
    return pl.pallas_call(
        _iou_pack_kernel,
        grid=(GRID1,),
        in_specs=[blk, blk, blk, smem, smem, smem],
        out_specs=[blk, blk, blk, blk],
        out_shape=[out, out, out, out],
    )(ps2, pe2, dp2, ls, le, dl)


def _sc_greedy_body(w0_h, w1_h, w2_h, w3_h, conf_h,
                    ev_idx_h, ev_conf_h, cnt_h,
                    w0_v, w1_v, w2_v, w3_v, conf_v,
                    ev_idx_v, ev_conf_v, cnt_v):
    first = (lax.axis_index("c") == 0) & (lax.axis_index("s") == 0)

    @pl.when(first)
    def _():
        pltpu.sync_copy(w0_h, w0_v)
        pltpu.sync_copy(w1_h, w1_v)
        pltpu.sync_copy(w2_h, w2_v)
        pltpu.sync_copy(w3_h, w3_v)
        pltpu.sync_copy(conf_h, conf_v)
        for k in range(8):
            ev_idx_v[pl.ds(k * 16, 16)] = jnp.full((16,), 2**30, jnp.int32)
            ev_conf_v[pl.ds(k * 16, 16)] = jnp.full((16,), 2.0, jnp.float32)

        lanes = lax.iota(jnp.int32, 16)

        def cond(st):
            g, _, _, _, _, _, cnt = st
            return (g < NG) & (cnt < M)

        lane0mask = lanes == 0

        def body(st):
            g, lane0, d0, d1, d2, d3, cnt = st
            a0 = w0_v[pl.ds(g * 16, 16)] & ~d0
            a1 = w1_v[pl.ds(g * 16, 16)] & ~d1
            a2 = w2_v[pl.ds(g * 16, 16)] & ~d2
            a3 = w3_v[pl.ds(g * 16, 16)] & ~d3
            avail = a0 | a1 | a2 | a3
            ok = (avail != 0) & (lanes >= lane0)
            lane = jnp.min(jnp.where(ok, lanes, jnp.int32(16)))
            hit = lane < 16

            sel = lanes == lane
            zv = jnp.zeros((16,), jnp.int32)
            x0 = jnp.sum(jnp.where(sel, a0, zv))
            x1 = jnp.sum(jnp.where(sel, a1, zv))
            x2 = jnp.sum(jnp.where(sel, a2, zv))
            x3 = jnp.sum(jnp.where(sel, a3, zv))
            t0 = x0 != 0
            t1 = (x1 != 0) & ~t0
            t2 = (x2 != 0) & ~(t0 | t1)
            t3 = (x3 != 0) & ~(t0 | t1 | t2)
            nd0 = jnp.where(t0, d0 | (x0 & (-x0)), d0)
            nd1 = jnp.where(t1, d1 | (x1 & (-x1)), d1)
            nd2 = jnp.where(t2, d2 | (x2 & (-x2)), d2)
            nd3 = jnp.where(t3, d3 | (x3 & (-x3)), d3)
            r = jnp.minimum(g * 16 + lane, jnp.int32(NPAD - 1))
            rv = jnp.full((16,), r, jnp.int32)
            cv = plsc.load_gather(conf_v, [rv])
            ev_slot = jnp.full((16,), cnt, jnp.int32)
            emask = lane0mask & (jnp.full((16,), lane, jnp.int32) < 16)
            plsc.store_scatter(ev_idx_v, [ev_slot], rv, mask=emask)
            plsc.store_scatter(ev_conf_v, [ev_slot], cv, mask=emask)

            ng = jnp.where(hit, g, g + 1)
            nlane0 = jnp.where(hit, lane + 1, jnp.int32(0))
            ncnt = jnp.where(hit, cnt + 1, cnt)
            return (ng, nlane0, nd0, nd1, nd2, nd3, ncnt)

        z = jnp.int32(0)
        st = lax.while_loop(cond, body, (z, z, z, z, z, z, z))
        cnt_v[...] = jnp.full((16,), st[6], jnp.int32)
        pltpu.sync_copy(ev_idx_v, ev_idx_h)
        pltpu.sync_copy(ev_conf_v, ev_conf_h)
        pltpu.sync_copy(cnt_v, cnt_h)


def _stage2(w0, w1, w2, w3, conf):
    mesh = plsc.VectorSubcoreMesh(core_axis_name="c", subcore_axis_name="s")
    return pl.kernel(
        _sc_greedy_body,
        out_type=[
            jax.ShapeDtypeStruct((M,), jnp.int32),
            jax.ShapeDtypeStruct((M,), jnp.float32),
            jax.ShapeDtypeStruct((16,), jnp.int32),
        ],
        mesh=mesh,
        compiler_params=pltpu.CompilerParams(needs_layout_passes=False),
        scratch_types=[
            pltpu.VMEM((NPAD,), jnp.int32),
            pltpu.VMEM((NPAD,), jnp.int32),
            pltpu.VMEM((NPAD,), jnp.int32),
            pltpu.VMEM((NPAD,), jnp.int32),
            pltpu.VMEM((NPAD,), jnp.float32),
            pltpu.VMEM((M,), jnp.int32),
            pltpu.VMEM((M,), jnp.float32),
            pltpu.VMEM((16,), jnp.int32),
        ],
    )(w0, w1, w2, w3, conf)


def _rank_ap_kernel(conf_ref, idx_ref, evc_ref, evi_ref,
                    vrow_ref, vcol_ref, ap_ref):
    conf = conf_ref[...]
    idx = idx_ref[...]
    lane = lax.broadcasted_iota(jnp.int32, (1, 128), 1)
    rank_row = jnp.zeros((1, 128), jnp.float32)
    for e in range(M):
        c = evc_ref[e]
        ie = evi_ref[e]
        gt = jnp.sum((conf > c).astype(jnp.float32))
        eqb = jnp.sum(((conf == c) & (idx < ie)).astype(jnp.float32))
        re = jnp.float32(1.0) + gt + eqb
        rank_row = rank_row + jnp.where(lane == e, re, jnp.float32(0.0))

    vrow = vrow_ref[...]
    vcol = vcol_ref[...]
    eye = (lax.broadcasted_iota(jnp.int32, (128, 128), 0) ==
           lax.broadcasted_iota(jnp.int32, (128, 128), 1)).astype(jnp.float32)
    rank_col = jnp.sum(eye * rank_row, axis=1, keepdims=True)
    less = ((rank_col < rank_row) & (vrow > 0.5)).astype(jnp.float32)
    lvl = jnp.float32(1.0) + jnp.sum(less, axis=0, keepdims=True)
    v = lvl / rank_row
    jrow = (lax.broadcasted_iota(jnp.int32, (128, 1), 0) + 1).astype(jnp.float32)
    a = jnp.where((lvl >= jrow) & (vcol > 0.5), v, jnp.float32(0.0))
    mx = jnp.max(a, axis=1, keepdims=True)
    ap_ref[...] = jnp.sum(mx, keepdims=True) * jnp.float32(1.0 / M)


def _stage3(conf2, idx2, ev_conf, ev_idx, vrow, vcol):
    smem = pl.BlockSpec(memory_space=pltpu.SMEM)
    return pl.pallas_call(
        _rank_ap_kernel,
        in_specs=[pl.BlockSpec((ROWS, 128), lambda: (0, 0)),
                  pl.BlockSpec((ROWS, 128), lambda: (0, 0)),
                  smem, smem,
                  pl.BlockSpec((128, 1), lambda: (0, 0)),
                  pl.BlockSpec((1, 128), lambda: (0, 0))],
        out_specs=pl.BlockSpec((1, 1), lambda: (0, 0)),
        out_shape=jax.ShapeDtypeStruct((1, 1), jnp.float32),
    )(conf2, idx2, ev_conf, ev_idx, vrow, vcol)


@jax.jit
def kernel(proposals, labels):
    conf = proposals[:, 0]
    ps = proposals[:, 1] / FPS
    pe = proposals[:, 2] / FPS
    dp = pe - ps
    ls = labels[:, 0]
    le = labels[:, 1]
    dl = le - ls

    pad = NPAD - N
    ps2 = jnp.concatenate([ps, jnp.full((pad,), 1e9, jnp.float32)]).reshape(ROWS, 128)
    pe2 = jnp.concatenate([pe, jnp.full((pad,), 1e9 + 1.0, jnp.float32)]).reshape(ROWS, 128)
    dp2 = jnp.concatenate([dp, jnp.full((pad,), 1.0, jnp.float32)]).reshape(ROWS, 128)
    conf_p = jnp.concatenate([conf, jnp.full((pad,), -1.0, jnp.float32)])

    w0, w1, w2, w3 = _stage1(ps2, pe2, dp2, ls, le, dl)

    ev_idx, ev_conf, cntv = _stage2(
        w0.reshape(NPAD), w1.reshape(NPAD), w2.reshape(NPAD), w3.reshape(NPAD),
        conf_p)

    cnt = cntv[0]
    valid = (jnp.arange(M, dtype=jnp.int32) < cnt).astype(jnp.float32)
    conf2 = conf_p.reshape(ROWS, 128)
    idx2 = jnp.arange(NPAD, dtype=jnp.int32).reshape(ROWS, 128)
    ap = _stage3(conf2, idx2, ev_conf, ev_idx,
                 valid.reshape(M, 1), valid.reshape(1, M))
    return ap[0, 0]


# speedup vs baseline: 2286.1591x; 1.6560x over previous
import functools

import jax
import jax.numpy as jnp
from jax import lax
from jax.experimental import pallas as pl
from jax.experimental.pallas import tpu as pltpu
from jax.experimental.pallas import tpu_sc as plsc

FPS = 25.0
N = 20000
M = 128
NPAD = 20480
ROWS = NPAD // 128
NG = NPAD // 16
NSG = NG // 16
BLK = 32
GRID1 = ROWS // BLK


def _iou_pack_kernel(ps_ref, pe_ref, dp_ref, ls_ref, le_ref, dl_ref,
                     w0_ref, w1_ref, w2_ref, w3_ref,
                     g0_ref, g1_ref, g2_ref, g3_ref):
    ps = ps_ref[...]
    pe = pe_ref[...]
    dp = dp_ref[...]
    seg = (lax.broadcasted_iota(jnp.int32, (128, 8), 0) // 16 ==
           lax.broadcasted_iota(jnp.int32, (128, 8), 1)).astype(jnp.float32)
    accs = [jnp.zeros((BLK, 128), jnp.int32) for _ in range(4)]
    gaccs = [jnp.zeros((BLK, 8), jnp.int32) for _ in range(4)]
    for j in range(M):
        ls = ls_ref[j]
        le = le_ref[j]
        dl = dl_ref[j]
        imin = jnp.maximum(ps, ls)
        imax = jnp.minimum(pe, le)
        inter = jnp.maximum(imax - imin, jnp.float32(0.0))
        union = dp + dl - inter
        tpb = (inter / union) > jnp.float32(0.5)
        tp = tpb.astype(jnp.int32)
        cnts = jnp.dot(tpb.astype(jnp.float32), seg,
                       preferred_element_type=jnp.float32)
        gtp = (cnts > jnp.float32(0.5)).astype(jnp.int32)
        w = j // 32
        b = j % 32
        accs[w] = accs[w] | (tp << b)
        gaccs[w] = gaccs[w] | (gtp << b)
    w0_ref[...] = accs[0]
    w1_ref[...] = accs[1]
    w2_ref[...] = accs[2]
    w3_ref[...] = accs[3]
    g0_ref[...] = gaccs[0]
    g1_ref[...] = gaccs[1]
    g2_ref[...] = gaccs[2]
    g3_ref[...] = gaccs[3]


def _stage1(ps2, pe2, dp2, ls, le, dl):
    blk = pl.BlockSpec((BLK, 128), lambda r: (r, 0))
    gblk = pl.BlockSpec((BLK, 8), lambda r: (r, 0))
    smem = pl.BlockSpec(memory_space=pltpu.SMEM)
    out = jax.ShapeDtypeStruct((ROWS, 128), jnp.int32)
    gout = jax.ShapeDtypeStruct((ROWS, 8), jnp.int32)
    return pl.pallas_call(
        _iou_pack_kernel,
        grid=(GRID1,),
        in_specs=[blk, blk, blk, smem, smem, smem],
        out_specs=[blk, blk, blk, blk, gblk, gblk, gblk, gblk],
        out_shape=[out, out, out, out, gout, gout, gout, gout],
    )(ps2, pe2, dp2, ls, le, dl)


def _sc_greedy_body(w0_h, w1_h, w2_h, w3_h, g0_h, g1_h, g2_h, g3_h, conf_h,
                    ev_idx_h, ev_conf_h, cnt_h,
                    w0_v, w1_v, w2_v, w3_v, g0_v, g1_v, g2_v, g3_v, conf_v,
                    ev_idx_v, ev_conf_v, cnt_v):
    first = (lax.axis_index("c") == 0) & (lax.axis_index("s") == 0)

    @pl.when(first)
    def _():
        pltpu.sync_copy(w0_h, w0_v)
        pltpu.sync_copy(w1_h, w1_v)
        pltpu.sync_copy(w2_h, w2_v)
        pltpu.sync_copy(w3_h, w3_v)
        pltpu.sync_copy(g0_h, g0_v)
        pltpu.sync_copy(g1_h, g1_v)
        pltpu.sync_copy(g2_h, g2_v)
        pltpu.sync_copy(g3_h, g3_v)
        pltpu.sync_copy(conf_h, conf_v)
        for k in range(8):
            ev_idx_v[pl.ds(k * 16, 16)] = jnp.full((16,), 2**30, jnp.int32)
            ev_conf_v[pl.ds(k * 16, 16)] = jnp.full((16,), 2.0, jnp.float32)

        lanes = lax.iota(jnp.int32, 16)
        lane0mask = lanes == 0
        zv = jnp.zeros((16,), jnp.int32)

        def cond(st):
            sg, _, _, _, _, cnt = st
            return (sg < NSG) & (cnt < M)

        def body(st):
            sg, d0, d1, d2, d3, cnt = st
            q0 = g0_v[pl.ds(sg * 16, 16)] & ~d0
            q1 = g1_v[pl.ds(sg * 16, 16)] & ~d1
            q2 = g2_v[pl.ds(sg * 16, 16)] & ~d2
            q3 = g3_v[pl.ds(sg * 16, 16)] & ~d3
            gav = (q0 | q1 | q2 | q3) != 0
            gsel = jnp.min(jnp.where(gav, lanes, jnp.int32(16)))
            ghit = gsel < 16
            g = jnp.minimum(sg * 16 + gsel, jnp.int32(NG - 1))

            raw0 = w0_v[pl.ds(g * 16, 16)]
            raw1 = w1_v[pl.ds(g * 16, 16)]
            raw2 = w2_v[pl.ds(g * 16, 16)]
            raw3 = w3_v[pl.ds(g * 16, 16)]
            avail = (raw0 & ~d0) | (raw1 & ~d1) | (raw2 & ~d2) | (raw3 & ~d3)
            lane = jnp.min(jnp.where(avail != 0, lanes, jnp.int32(16)))
            r = jnp.minimum(g * 16 + lane, jnp.int32(NPAD - 1))
            rv = jnp.full((16,), r, jnp.int32)

            x0 = plsc.load_gather(w0_v, [rv]) & ~d0
            x1 = plsc.load_gather(w1_v, [rv]) & ~d1
            x2 = plsc.load_gather(w2_v, [rv]) & ~d2
            x3 = plsc.load_gather(w3_v, [rv]) & ~d3
            gselv = jnp.full((16,), gsel, jnp.int32) < 16
            lanev = jnp.full((16,), lane, jnp.int32) < 16
            hitv = gselv & lanev
            stalev = gselv & ~lanev
            t0 = (x0 != 0) & hitv
            t1 = (x1 != 0) & ~(x0 != 0) & hitv
            t2 = (x2 != 0) & ~((x0 != 0) | (x1 != 0)) & hitv
            t3 = (x3 != 0) & ~((x0 != 0) | (x1 != 0) | (x2 != 0)) & hitv
            nd0 = jnp.where(t0, d0 | (x0 & (zv - x0)), d0)
            nd1 = jnp.where(t1, d1 | (x1 & (zv - x1)), d1)
            nd2 = jnp.where(t2, d2 | (x2 & (zv - x2)), d2)
            nd3 = jnp.where(t3, d3 | (x3 & (zv - x3)), d3)

            cv = plsc.load_gather(conf_v, [rv])
            ev_slot = jnp.full((16,), cnt, jnp.int32)
            emask = lane0mask & hitv
            plsc.store_scatter(ev_idx_v, [ev_slot], rv, mask=emask)
            plsc.store_scatter(ev_conf_v, [ev_slot], cv, mask=emask)

            plsc.store_scatter(w0_v, [rv], zv, mask=emask)
            plsc.store_scatter(w1_v, [rv], zv, mask=emask)
            plsc.store_scatter(w2_v, [rv], zv, mask=emask)
            plsc.store_scatter(w3_v, [rv], zv, mask=emask)
            gv = jnp.full((16,), g, jnp.int32)
            smask = lane0mask & stalev
            plsc.store_scatter(g0_v, [gv], plsc.load_gather(g0_v, [gv]) & d0, mask=smask)
            plsc.store_scatter(g1_v, [gv], plsc.load_gather(g1_v, [gv]) & d1, mask=smask)
            plsc.store_scatter(g2_v, [gv], plsc.load_gather(g2_v, [gv]) & d2, mask=smask)
            plsc.store_scatter(g3_v, [gv], plsc.load_gather(g3_v, [gv]) & d3, mask=smask)

            hit_ev = ghit & (lane < 16)
            nsg = jnp.where(ghit, sg, sg + 1)
            ncnt = jnp.where(hit_ev, cnt + 1, cnt)
            return (nsg, nd0, nd1, nd2, nd3, ncnt)

        z = jnp.int32(0)
        st = lax.while_loop(cond, body, (z, zv, zv, zv, zv, z))
        cnt_v[...] = jnp.full((16,), st[5], jnp.int32)
        pltpu.sync_copy(ev_idx_v, ev_idx_h)
        pltpu.sync_copy(ev_conf_v, ev_conf_h)
        pltpu.sync_copy(cnt_v, cnt_h)


def _stage2(w0, w1, w2, w3, g0, g1, g2, g3, conf):
    mesh = plsc.VectorSubcoreMesh(core_axis_name="c", subcore_axis_name="s")
    return pl.kernel(
        _sc_greedy_body,
        out_type=[
            jax.ShapeDtypeStruct((M,), jnp.int32),
            jax.ShapeDtypeStruct((M,), jnp.float32),
            jax.ShapeDtypeStruct((16,), jnp.int32),
        ],
        mesh=mesh,
        compiler_params=pltpu.CompilerParams(needs_layout_passes=False),
        scratch_types=[
            pltpu.VMEM((NPAD,), jnp.int32),
            pltpu.VMEM((NPAD,), jnp.int32),
            pltpu.VMEM((NPAD,), jnp.int32),
            pltpu.VMEM((NPAD,), jnp.int32),
            pltpu.VMEM((NG,), jnp.int32),
            pltpu.VMEM((NG,), jnp.int32),
            pltpu.VMEM((NG,), jnp.int32),
            pltpu.VMEM((NG,), jnp.int32),
            pltpu.VMEM((NPAD,), jnp.float32),
            pltpu.VMEM((M,), jnp.int32),
            pltpu.VMEM((M,), jnp.float32),
            pltpu.VMEM((16,), jnp.int32),
        ],
    )(w0, w1, w2, w3, g0, g1, g2, g3, conf)


def _rank_ap_kernel(conf_ref, evc_ref, evi_ref, cnt_ref, ap_ref):
    conf = conf_ref[...]
    idx = (lax.broadcasted_iota(jnp.int32, (ROWS, 128), 0) * 128 +
           lax.broadcasted_iota(jnp.int32, (ROWS, 128), 1))
    lane = lax.broadcasted_iota(jnp.int32, (1, 128), 1)
    rank_row = jnp.zeros((1, 128), jnp.float32)
    for e in range(M):
        c = evc_ref[e]
        ie = evi_ref[e]
        gt = jnp.sum((conf > c).astype(jnp.float32))
        eqb = jnp.sum(((conf == c) & (idx < ie)).astype(jnp.float32))
        re = jnp.float32(1.0) + gt + eqb
        rank_row = rank_row + jnp.where(lane == e, re, jnp.float32(0.0))

    cnt = cnt_ref[0]
    vcol = lane < cnt
    vrow = lax.broadcasted_iota(jnp.int32, (128, 1), 0) < cnt
    eye = (lax.broadcasted_iota(jnp.int32, (128, 128), 0) ==
           lax.broadcasted_iota(jnp.int32, (128, 128), 1)).astype(jnp.float32)
    rank_col = jnp.sum(eye * rank_row, axis=1, keepdims=True)
    less = ((rank_col < rank_row) & vrow).astype(jnp.float32)
    lvl = jnp.float32(1.0) + jnp.sum(less, axis=0, keepdims=True)
    v = lvl / rank_row
    jrow = (lax.broadcasted_iota(jnp.int32, (128, 1), 0) + 1).astype(jnp.float32)
    a = jnp.where((lvl >= jrow) & vcol, v, jnp.float32(0.0))
    mx = jnp.max(a, axis=1, keepdims=True)
    ap_ref[...] = jnp.sum(mx, keepdims=True) * jnp.float32(1.0 / M)


def _stage3(conf2, ev_conf, ev_idx, cntv):
    smem = pl.BlockSpec(memory_space=pltpu.SMEM)
    return pl.pallas_call(
        _rank_ap_kernel,
        in_specs=[pl.BlockSpec((ROWS, 128), lambda: (0, 0)),
                  smem, smem, smem],
        out_specs=pl.BlockSpec((1, 1), lambda: (0, 0)),
        out_shape=jax.ShapeDtypeStruct((1, 1), jnp.float32),
    )(conf2, ev_conf, ev_idx, cntv)


@jax.jit
def kernel(proposals, labels):
    conf = proposals[:, 0]
    ps = proposals[:, 1] / FPS
    pe = proposals[:, 2] / FPS
    dp = pe - ps
    ls = labels[:, 0]
    le = labels[:, 1]
    dl = le - ls

    pad = NPAD - N
    ps2 = jnp.concatenate([ps, jnp.full((pad,), 1e9, jnp.float32)]).reshape(ROWS, 128)
    pe2 = jnp.concatenate([pe, jnp.full((pad,), 1e9 + 1.0, jnp.float32)]).reshape(ROWS, 128)
    dp2 = jnp.concatenate([dp, jnp.full((pad,), 1.0, jnp.float32)]).reshape(ROWS, 128)
    conf_p = jnp.concatenate([conf, jnp.full((pad,), -1.0, jnp.float32)])

    w0, w1, w2, w3, g0, g1, g2, g3 = _stage1(ps2, pe2, dp2, ls, le, dl)

    ev_idx, ev_conf, cntv = _stage2(
        w0.reshape(NPAD), w1.reshape(NPAD), w2.reshape(NPAD), w3.reshape(NPAD),
        g0.reshape(NG), g1.reshape(NG), g2.reshape(NG), g3.reshape(NG),
        conf_p)

    ap = _stage3(conf_p.reshape(ROWS, 128), ev_conf, ev_idx, cntv)
    return ap[0, 0]


# speedup vs baseline: 2321.3018x; 1.0154x over previous
import functools

import jax
import jax.numpy as jnp
from jax import lax
from jax.experimental import pallas as pl
from jax.experimental.pallas import tpu as pltpu
from jax.experimental.pallas import tpu_sc as plsc

FPS = 25.0
N = 20000
M = 128
NPAD = 20480
ROWS = NPAD // 128
NG = NPAD // 16
NSG = NG // 16
BLK = 160
GRID1 = ROWS // BLK


def _iou_pack_kernel(ps_ref, pe_ref, dp_ref, ls_ref, le_ref, dl_ref,
                     w0_ref, w1_ref, w2_ref, w3_ref,
                     g0_ref, g1_ref, g2_ref, g3_ref):
    ps = ps_ref[...]
    pe = pe_ref[...]
    dp = dp_ref[...]
    seg = (lax.broadcasted_iota(jnp.int32, (128, 8), 0) // 16 ==
           lax.broadcasted_iota(jnp.int32, (128, 8), 1)).astype(jnp.float32)
    accs = [jnp.zeros((BLK, 128), jnp.int32) for _ in range(4)]
    gaccs = [jnp.zeros((BLK, 8), jnp.int32) for _ in range(4)]
    for j in range(M):
        ls = ls_ref[j]
        le = le_ref[j]
        dl = dl_ref[j]
        imin = jnp.maximum(ps, ls)
        imax = jnp.minimum(pe, le)
        inter = jnp.maximum(imax - imin, jnp.float32(0.0))
        union = dp + dl - inter
        tpb = (inter / union) > jnp.float32(0.5)
        tp = tpb.astype(jnp.int32)
        cnts = jnp.dot(tpb.astype(jnp.float32), seg,
                       preferred_element_type=jnp.float32)
        gtp = (cnts > jnp.float32(0.5)).astype(jnp.int32)
        w = j // 32
        b = j % 32
        accs[w] = accs[w] | (tp << b)
        gaccs[w] = gaccs[w] | (gtp << b)
    w0_ref[...] = accs[0]
    w1_ref[...] = accs[1]
    w2_ref[...] = accs[2]
    w3_ref[...] = accs[3]
    g0_ref[...] = gaccs[0]
    g1_ref[...] = gaccs[1]
    g2_ref[...] = gaccs[2]
    g3_ref[...] = gaccs[3]


def _stage1(ps2, pe2, dp2, ls, le, dl):
    blk = pl.BlockSpec((BLK, 128), lambda r: (r, 0))
    gblk = pl.BlockSpec((BLK, 8), lambda r: (r, 0))
    smem = pl.BlockSpec(memory_space=pltpu.SMEM)
    out = jax.ShapeDtypeStruct((ROWS, 128), jnp.int32)
    gout = jax.ShapeDtypeStruct((ROWS, 8), jnp.int32)
    return pl.pallas_call(
        _iou_pack_kernel,
        grid=(GRID1,),
        in_specs=[blk, blk, blk, smem, smem, smem],
        out_specs=[blk, blk, blk, blk, gblk, gblk, gblk, gblk],
        out_shape=[out, out, out, out, gout, gout, gout, gout],
    )(ps2, pe2, dp2, ls, le, dl)


def _sc_greedy_body(w0_h, w1_h, w2_h, w3_h, g0_h, g1_h, g2_h, g3_h, conf_h,
                    ev_idx_h, ev_conf_h, cnt_h,
                    w0_v, w1_v, w2_v, w3_v, g0_v, g1_v, g2_v, g3_v, conf_v,
                    ev_idx_v, ev_conf_v, cnt_v):
    first = (lax.axis_index("c") == 0) & (lax.axis_index("s") == 0)

    @pl.when(first)
    def _():
        pltpu.sync_copy(w0_h, w0_v)
        pltpu.sync_copy(w1_h, w1_v)
        pltpu.sync_copy(w2_h, w2_v)
        pltpu.sync_copy(w3_h, w3_v)
        pltpu.sync_copy(g0_h, g0_v)
        pltpu.sync_copy(g1_h, g1_v)
        pltpu.sync_copy(g2_h, g2_v)
        pltpu.sync_copy(g3_h, g3_v)
        pltpu.sync_copy(conf_h, conf_v)
        for k in range(8):
            ev_idx_v[pl.ds(k * 16, 16)] = jnp.full((16,), 2**30, jnp.int32)
            ev_conf_v[pl.ds(k * 16, 16)] = jnp.full((16,), 2.0, jnp.float32)

        lanes = lax.iota(jnp.int32, 16)
        lane0mask = lanes == 0
        zv = jnp.zeros((16,), jnp.int32)

        def cond(st):
            sg, _, _, _, _, cnt = st
            return (sg < NSG) & (cnt < M)

        def body(st):
            sg, d0, d1, d2, d3, cnt = st
            q0 = g0_v[pl.ds(sg * 16, 16)] & ~d0
            q1 = g1_v[pl.ds(sg * 16, 16)] & ~d1
            q2 = g2_v[pl.ds(sg * 16, 16)] & ~d2
            q3 = g3_v[pl.ds(sg * 16, 16)] & ~d3
            gav = (q0 | q1 | q2 | q3) != 0
            ghit = jnp.any(gav)
            gsel = plsc.all_reduce_ffs(gav)[0]
            g = jnp.minimum(sg * 16 + gsel, jnp.int32(NG - 1))

            raw0 = w0_v[pl.ds(g * 16, 16)]
            raw1 = w1_v[pl.ds(g * 16, 16)]
            raw2 = w2_v[pl.ds(g * 16, 16)]
            raw3 = w3_v[pl.ds(g * 16, 16)]
            avm = ((raw0 & ~d0) | (raw1 & ~d1) | (raw2 & ~d2) | (raw3 & ~d3)) != 0
            lhit = jnp.any(avm)
            lane = plsc.all_reduce_ffs(avm)[0]
            r = jnp.minimum(g * 16 + lane, jnp.int32(NPAD - 1))
            rv = jnp.full((16,), r, jnp.int32)

            x0 = plsc.load_gather(w0_v, [rv]) & ~d0
            x1 = plsc.load_gather(w1_v, [rv]) & ~d1
            x2 = plsc.load_gather(w2_v, [rv]) & ~d2
            x3 = plsc.load_gather(w3_v, [rv]) & ~d3
            gsel_i = jnp.where(ghit, jnp.int32(1), jnp.int32(0))
            lane_i = jnp.where(lhit, jnp.int32(1), jnp.int32(0))
            gselv = jnp.full((16,), gsel_i, jnp.int32) == 1
            lanev = jnp.full((16,), lane_i, jnp.int32) == 1
            hitv = gselv & lanev
            stalev = gselv & ~lanev
            t0 = (x0 != 0) & hitv
            t1 = (x1 != 0) & ~(x0 != 0) & hitv
            t2 = (x2 != 0) & ~((x0 != 0) | (x1 != 0)) & hitv
            t3 = (x3 != 0) & ~((x0 != 0) | (x1 != 0) | (x2 != 0)) & hitv
            nd0 = jnp.where(t0, d0 | (x0 & (zv - x0)), d0)
            nd1 = jnp.where(t1, d1 | (x1 & (zv - x1)), d1)
            nd2 = jnp.where(t2, d2 | (x2 & (zv - x2)), d2)
            nd3 = jnp.where(t3, d3 | (x3 & (zv - x3)), d3)

            cv = plsc.load_gather(conf_v, [rv])
            ev_slot = jnp.full((16,), cnt, jnp.int32)
            emask = lane0mask & hitv
            plsc.store_scatter(ev_idx_v, [ev_slot], rv, mask=emask)
            plsc.store_scatter(ev_conf_v, [ev_slot], cv, mask=emask)

            plsc.store_scatter(w0_v, [rv], zv, mask=emask)
            plsc.store_scatter(w1_v, [rv], zv, mask=emask)
            plsc.store_scatter(w2_v, [rv], zv, mask=emask)
            plsc.store_scatter(w3_v, [rv], zv, mask=emask)
            gv = jnp.full((16,), g, jnp.int32)
            smask = lane0mask & stalev
            plsc.store_scatter(g0_v, [gv], plsc.load_gather(g0_v, [gv]) & d0, mask=smask)
            plsc.store_scatter(g1_v, [gv], plsc.load_gather(g1_v, [gv]) & d1, mask=smask)
            plsc.store_scatter(g2_v, [gv], plsc.load_gather(g2_v, [gv]) & d2, mask=smask)
            plsc.store_scatter(g3_v, [gv], plsc.load_gather(g3_v, [gv]) & d3, mask=smask)

            hit_ev = ghit & lhit
            nsg = jnp.where(ghit, sg, sg + 1)
            ncnt = jnp.where(hit_ev, cnt + 1, cnt)
            return (nsg, nd0, nd1, nd2, nd3, ncnt)

        z = jnp.int32(0)
        st = lax.while_loop(cond, body, (z, zv, zv, zv, zv, z))
        cnt_v[...] = jnp.full((16,), st[5], jnp.int32)
        pltpu.sync_copy(ev_idx_v, ev_idx_h)
        pltpu.sync_copy(ev_conf_v, ev_conf_h)
        pltpu.sync_copy(cnt_v, cnt_h)


def _stage2(w0, w1, w2, w3, g0, g1, g2, g3, conf):
    mesh = plsc.VectorSubcoreMesh(core_axis_name="c", subcore_axis_name="s")
    return pl.kernel(
        _sc_greedy_body,
        out_type=[
            jax.ShapeDtypeStruct((M,), jnp.int32),
            jax.ShapeDtypeStruct((M,), jnp.float32),
            jax.ShapeDtypeStruct((16,), jnp.int32),
        ],
        mesh=mesh,
        compiler_params=pltpu.CompilerParams(needs_layout_passes=False),
        scratch_types=[
            pltpu.VMEM((NPAD,), jnp.int32),
            pltpu.VMEM((NPAD,), jnp.int32),
            pltpu.VMEM((NPAD,), jnp.int32),
            pltpu.VMEM((NPAD,), jnp.int32),
            pltpu.VMEM((NG,), jnp.int32),
            pltpu.VMEM((NG,), jnp.int32),
            pltpu.VMEM((NG,), jnp.int32),
            pltpu.VMEM((NG,), jnp.int32),
            pltpu.VMEM((NPAD,), jnp.float32),
            pltpu.VMEM((M,), jnp.int32),
            pltpu.VMEM((M,), jnp.float32),
            pltpu.VMEM((16,), jnp.int32),
        ],
    )(w0, w1, w2, w3, g0, g1, g2, g3, conf)


def _rank_ap_kernel(conf_ref, evc_ref, evi_ref, cnt_ref, ap_ref):
    conf = conf_ref[...]
    idx = (lax.broadcasted_iota(jnp.int32, (ROWS, 128), 0) * 128 +
           lax.broadcasted_iota(jnp.int32, (ROWS, 128), 1))
    lane = lax.broadcasted_iota(jnp.int32, (1, 128), 1)
    rank_row = jnp.zeros((1, 128), jnp.float32)
    for e in range(M):
        c = evc_ref[e]
        ie = evi_ref[e]
        gt = jnp.sum((conf > c).astype(jnp.float32))
        eqb = jnp.sum(((conf == c) & (idx < ie)).astype(jnp.float32))
        re = jnp.float32(1.0) + gt + eqb
        rank_row = rank_row + jnp.where(lane == e, re, jnp.float32(0.0))

    cnt = cnt_ref[0]
    vcol = lane < cnt
    vrow = lax.broadcasted_iota(jnp.int32, (128, 1), 0) < cnt
    eye = (lax.broadcasted_iota(jnp.int32, (128, 128), 0) ==
           lax.broadcasted_iota(jnp.int32, (128, 128), 1)).astype(jnp.float32)
    rank_col = jnp.sum(eye * rank_row, axis=1, keepdims=True)
    less = ((rank_col < rank_row) & vrow).astype(jnp.float32)
    lvl = jnp.float32(1.0) + jnp.sum(less, axis=0, keepdims=True)
    v = lvl / rank_row
    jrow = (lax.broadcasted_iota(jnp.int32, (128, 1), 0) + 1).astype(jnp.float32)
    a = jnp.where((lvl >= jrow) & vcol, v, jnp.float32(0.0))
    mx = jnp.max(a, axis=1, keepdims=True)
    ap_ref[...] = jnp.sum(mx, keepdims=True) * jnp.float32(1.0 / M)


def _stage3(conf2, ev_conf, ev_idx, cntv):
    smem = pl.BlockSpec(memory_space=pltpu.SMEM)
    return pl.pallas_call(
        _rank_ap_kernel,
        in_specs=[pl.BlockSpec((ROWS, 128), lambda: (0, 0)),
                  smem, smem, smem],
        out_specs=pl.BlockSpec((1, 1), lambda: (0, 0)),
        out_shape=jax.ShapeDtypeStruct((1, 1), jnp.float32),
    )(conf2, ev_conf, ev_idx, cntv)


@jax.jit
def kernel(proposals, labels):
    conf = proposals[:, 0]
    ps = proposals[:, 1] / FPS
    pe = proposals[:, 2] / FPS
    dp = pe - ps
    ls = labels[:, 0]
    le = labels[:, 1]
    dl = le - ls

    pad = NPAD - N
    ps2 = jnp.concatenate([ps, jnp.full((pad,), 1e9, jnp.float32)]).reshape(ROWS, 128)
    pe2 = jnp.concatenate([pe, jnp.full((pad,), 1e9 + 1.0, jnp.float32)]).reshape(ROWS, 128)
    dp2 = jnp.concatenate([dp, jnp.full((pad,), 1.0, jnp.float32)]).reshape(ROWS, 128)
    conf_p = jnp.concatenate([conf, jnp.full((pad,), -1.0, jnp.float32)])

    w0, w1, w2, w3, g0, g1, g2, g3 = _stage1(ps2, pe2, dp2, ls, le, dl)

    ev_idx, ev_conf, cntv = _stage2(
        w0.reshape(NPAD), w1.reshape(NPAD), w2.reshape(NPAD), w3.reshape(NPAD),
        g0.reshape(NG), g1.reshape(NG), g2.reshape(NG), g3.reshape(NG),
        conf_p)

    ap = _stage3(conf_p.reshape(ROWS, 128), ev_conf, ev_idx, cntv)
    return ap[0, 0]


# speedup vs baseline: 2521.7028x; 1.0863x over previous
import functools

import jax
import jax.numpy as jnp
from jax import lax
from jax.experimental import pallas as pl
from jax.experimental.pallas import tpu as pltpu
from jax.experimental.pallas import tpu_sc as plsc

FPS = 25.0
N = 20000
M = 128
NPAD = 20480
ROWS = NPAD // 128
NG = NPAD // 16
NSG = NG // 16
BLK = 160
GRID1 = ROWS // BLK


def _iou_pack_kernel(ps_ref, pe_ref, dp_ref, ls_ref, le_ref, dl_ref,
                     w0_ref, w1_ref, w2_ref, w3_ref,
                     g0_ref, g1_ref, g2_ref, g3_ref):
    ps = ps_ref[...]
    pe = pe_ref[...]
    dp = dp_ref[...]
    seg = (lax.broadcasted_iota(jnp.int32, (128, 8), 0) // 16 ==
           lax.broadcasted_iota(jnp.int32, (128, 8), 1)).astype(jnp.float32)
    accs = [jnp.zeros((BLK, 128), jnp.int32) for _ in range(4)]
    gaccs = [jnp.zeros((BLK, 8), jnp.int32) for _ in range(4)]
    for j in range(M):
        ls = ls_ref[j]
        le = le_ref[j]
        dl = dl_ref[j]
        imin = jnp.maximum(ps, ls)
        imax = jnp.minimum(pe, le)
        inter = jnp.maximum(imax - imin, jnp.float32(0.0))
        union = dp + dl - inter
        tpb = (inter / union) > jnp.float32(0.5)
        tp = tpb.astype(jnp.int32)
        cnts = jnp.dot(tpb.astype(jnp.float32), seg,
                       preferred_element_type=jnp.float32)
        gtp = (cnts > jnp.float32(0.5)).astype(jnp.int32)
        w = j // 32
        b = j % 32
        accs[w] = accs[w] | (tp << b)
        gaccs[w] = gaccs[w] | (gtp << b)
    w0_ref[...] = accs[0]
    w1_ref[...] = accs[1]
    w2_ref[...] = accs[2]
    w3_ref[...] = accs[3]
    g0_ref[...] = gaccs[0]
    g1_ref[...] = gaccs[1]
    g2_ref[...] = gaccs[2]
    g3_ref[...] = gaccs[3]


def _stage1(ps2, pe2, dp2, ls, le, dl):
    blk = pl.BlockSpec((BLK, 128), lambda r: (r, 0))
    gblk = pl.BlockSpec((BLK, 8), lambda r: (r, 0))
    smem = pl.BlockSpec(memory_space=pltpu.SMEM)
    out = jax.ShapeDtypeStruct((ROWS, 128), jnp.int32)
    gout = jax.ShapeDtypeStruct((ROWS, 8), jnp.int32)
    return pl.pallas_call(
        _iou_pack_kernel,
        grid=(GRID1,),
        in_specs=[blk, blk, blk, smem, smem, smem],
        out_specs=[blk, blk, blk, blk, gblk, gblk, gblk, gblk],
        out_shape=[out, out, out, out, gout, gout, gout, gout],
    )(ps2, pe2, dp2, ls, le, dl)


def _sc_greedy_body(w0_h, w1_h, w2_h, w3_h, g0_h, g1_h, g2_h, g3_h, conf_h,
                    ev_idx_h, ev_conf_h, cnt_h,
                    w0_v, w1_v, w2_v, w3_v, g0_v, g1_v, g2_v, g3_v, conf_v,
                    ev_idx_v, ev_conf_v, cnt_v):
    first = (lax.axis_index("c") == 0) & (lax.axis_index("s") == 0)

    @pl.when(first)
    def _():
        pltpu.sync_copy(w0_h, w0_v)
        pltpu.sync_copy(w1_h, w1_v)
        pltpu.sync_copy(w2_h, w2_v)
        pltpu.sync_copy(w3_h, w3_v)
        pltpu.sync_copy(g0_h, g0_v)
        pltpu.sync_copy(g1_h, g1_v)
        pltpu.sync_copy(g2_h, g2_v)
        pltpu.sync_copy(g3_h, g3_v)
        pltpu.sync_copy(conf_h, conf_v)
        for k in range(8):
            ev_idx_v[pl.ds(k * 16, 16)] = jnp.full((16,), 2**30, jnp.int32)
            ev_conf_v[pl.ds(k * 16, 16)] = jnp.full((16,), 2.0, jnp.float32)

        lanes = lax.iota(jnp.int32, 16)
        lane0mask = lanes == 0
        zv = jnp.zeros((16,), jnp.int32)

        def cond(st):
            sg, _, _, _, _, cnt = st
            return (sg < NSG) & (cnt < M)

        def body(st):
            sg, d0, d1, d2, d3, cnt = st
            q0 = g0_v[pl.ds(sg * 16, 16)] & ~d0
            q1 = g1_v[pl.ds(sg * 16, 16)] & ~d1
            q2 = g2_v[pl.ds(sg * 16, 16)] & ~d2
            q3 = g3_v[pl.ds(sg * 16, 16)] & ~d3
            gav = (q0 | q1 | q2 | q3) != 0
            gselb = plsc.all_reduce_population_count(gav) > 0
            gsel_v = plsc.all_reduce_ffs(gav)
            g_v = jnp.clip(sg * 16 + gsel_v, jnp.int32(0), jnp.int32(NG - 1))

            ridx = g_v * 16 + lanes
            raw0 = plsc.load_gather(w0_v, [ridx])
            raw1 = plsc.load_gather(w1_v, [ridx])
            raw2 = plsc.load_gather(w2_v, [ridx])
            raw3 = plsc.load_gather(w3_v, [ridx])
            avm = ((raw0 & ~d0) | (raw1 & ~d1) | (raw2 & ~d2) | (raw3 & ~d3)) != 0
            laneb = plsc.all_reduce_population_count(avm) > 0
            lane_v = plsc.all_reduce_ffs(avm)
            rv = jnp.clip(g_v * 16 + lane_v, jnp.int32(0), jnp.int32(NPAD - 1))

            x0 = plsc.load_gather(w0_v, [rv]) & ~d0
            x1 = plsc.load_gather(w1_v, [rv]) & ~d1
            x2 = plsc.load_gather(w2_v, [rv]) & ~d2
            x3 = plsc.load_gather(w3_v, [rv]) & ~d3
            hitv = gselb & laneb
            stalev = gselb & ~laneb
            t0 = (x0 != 0) & hitv
            t1 = (x1 != 0) & ~(x0 != 0) & hitv
            t2 = (x2 != 0) & ~((x0 != 0) | (x1 != 0)) & hitv
            t3 = (x3 != 0) & ~((x0 != 0) | (x1 != 0) | (x2 != 0)) & hitv
            nd0 = jnp.where(t0, d0 | (x0 & (zv - x0)), d0)
            nd1 = jnp.where(t1, d1 | (x1 & (zv - x1)), d1)
            nd2 = jnp.where(t2, d2 | (x2 & (zv - x2)), d2)
            nd3 = jnp.where(t3, d3 | (x3 & (zv - x3)), d3)

            cv = plsc.load_gather(conf_v, [rv])
            ev_slot = jnp.full((16,), cnt, jnp.int32)
            emask = lane0mask & hitv
            plsc.store_scatter(ev_idx_v, [ev_slot], rv, mask=emask)
            plsc.store_scatter(ev_conf_v, [ev_slot], cv, mask=emask)

            plsc.store_scatter(w0_v, [rv], zv, mask=emask)
            plsc.store_scatter(w1_v, [rv], zv, mask=emask)
            plsc.store_scatter(w2_v, [rv], zv, mask=emask)
            plsc.store_scatter(w3_v, [rv], zv, mask=emask)
            smask = lane0mask & stalev
            plsc.store_scatter(g0_v, [g_v], plsc.load_gather(g0_v, [g_v]) & d0, mask=smask)
            plsc.store_scatter(g1_v, [g_v], plsc.load_gather(g1_v, [g_v]) & d1, mask=smask)
            plsc.store_scatter(g2_v, [g_v], plsc.load_gather(g2_v, [g_v]) & d2, mask=smask)
            plsc.store_scatter(g3_v, [g_v], plsc.load_gather(g3_v, [g_v]) & d3, mask=smask)

            comb_v = (jnp.where(gselb, jnp.int32(2), jnp.int32(0)) |
                      jnp.where(laneb, jnp.int32(1), jnp.int32(0)))
            comb = comb_v[0]
            ghit = comb >= 2
            hit_ev = comb == 3
            nsg = jnp.where(ghit, sg, sg + 1)
            ncnt = jnp.where(hit_ev, cnt + 1, cnt)
            return (nsg, nd0, nd1, nd2, nd3, ncnt)

        z = jnp.int32(0)
        st = lax.while_loop(cond, body, (z, zv, zv, zv, zv, z))
        cnt_v[...] = jnp.full((16,), st[5], jnp.int32)
        pltpu.sync_copy(ev_idx_v, ev_idx_h)
        pltpu.sync_copy(ev_conf_v, ev_conf_h)
        pltpu.sync_copy(cnt_v, cnt_h)


def _stage2(w0, w1, w2, w3, g0, g1, g2, g3, conf):
    mesh = plsc.VectorSubcoreMesh(core_axis_name="c", subcore_axis_name="s")
    return pl.kernel(
        _sc_greedy_body,
        out_type=[
            jax.ShapeDtypeStruct((M,), jnp.int32),
            jax.ShapeDtypeStruct((M,), jnp.float32),
            jax.ShapeDtypeStruct((16,), jnp.int32),
        ],
        mesh=mesh,
        compiler_params=pltpu.CompilerParams(needs_layout_passes=False),
        scratch_types=[
            pltpu.VMEM((NPAD,), jnp.int32),
            pltpu.VMEM((NPAD,), jnp.int32),
            pltpu.VMEM((NPAD,), jnp.int32),
            pltpu.VMEM((NPAD,), jnp.int32),
            pltpu.VMEM((NG,), jnp.int32),
            pltpu.VMEM((NG,), jnp.int32),
            pltpu.VMEM((NG,), jnp.int32),
            pltpu.VMEM((NG,), jnp.int32),
            pltpu.VMEM((NPAD,), jnp.float32),
            pltpu.VMEM((M,), jnp.int32),
            pltpu.VMEM((M,), jnp.float32),
            pltpu.VMEM((16,), jnp.int32),
        ],
    )(w0, w1, w2, w3, g0, g1, g2, g3, conf)


def _rank_ap_kernel(conf_ref, evc_ref, evi_ref, cnt_ref, ap_ref):
    conf = conf_ref[...]
    idx = (lax.broadcasted_iota(jnp.int32, (ROWS, 128), 0) * 128 +
           lax.broadcasted_iota(jnp.int32, (ROWS, 128), 1))
    lane = lax.broadcasted_iota(jnp.int32, (1, 128), 1)
    rank_row = jnp.zeros((1, 128), jnp.float32)
    for e in range(M):
        c = evc_ref[e]
        ie = evi_ref[e]
        gt = jnp.sum((conf > c).astype(jnp.float32))
        eqb = jnp.sum(((conf == c) & (idx < ie)).astype(jnp.float32))
        re = jnp.float32(1.0) + gt + eqb
        rank_row = rank_row + jnp.where(lane == e, re, jnp.float32(0.0))

    cnt = cnt_ref[0]
    vcol = lane < cnt
    vrow = lax.broadcasted_iota(jnp.int32, (128, 1), 0) < cnt
    eye = (lax.broadcasted_iota(jnp.int32, (128, 128), 0) ==
           lax.broadcasted_iota(jnp.int32, (128, 128), 1)).astype(jnp.float32)
    rank_col = jnp.sum(eye * rank_row, axis=1, keepdims=True)
    less = ((rank_col < rank_row) & vrow).astype(jnp.float32)
    lvl = jnp.float32(1.0) + jnp.sum(less, axis=0, keepdims=True)
    v = lvl / rank_row
    jrow = (lax.broadcasted_iota(jnp.int32, (128, 1), 0) + 1).astype(jnp.float32)
    a = jnp.where((lvl >= jrow) & vcol, v, jnp.float32(0.0))
    mx = jnp.max(a, axis=1, keepdims=True)
    ap_ref[...] = jnp.sum(mx, keepdims=True) * jnp.float32(1.0 / M)


def _stage3(conf2, ev_conf, ev_idx, cntv):
    smem = pl.BlockSpec(memory_space=pltpu.SMEM)
    return pl.pallas_call(
        _rank_ap_kernel,
        in_specs=[pl.BlockSpec((ROWS, 128), lambda: (0, 0)),
                  smem, smem, smem],
        out_specs=pl.BlockSpec((1, 1), lambda: (0, 0)),
        out_shape=jax.ShapeDtypeStruct((1, 1), jnp.float32),
    )(conf2, ev_conf, ev_idx, cntv)


@jax.jit
def kernel(proposals, labels):
    conf = proposals[:, 0]
    ps = proposals[:, 1] / FPS
    pe = proposals[:, 2] / FPS
    dp = pe - ps
    ls = labels[:, 0]
    le = labels[:, 1]
    dl = le - ls

    pad = NPAD - N
    ps2 = jnp.concatenate([ps, jnp.full((pad,), 1e9, jnp.float32)]).reshape(ROWS, 128)
    pe2 = jnp.concatenate([pe, jnp.full((pad,), 1e9 + 1.0, jnp.float32)]).reshape(ROWS, 128)
    dp2 = jnp.concatenate([dp, jnp.full((pad,), 1.0, jnp.float32)]).reshape(ROWS, 128)
    conf_p = jnp.concatenate([conf, jnp.full((pad,), -1.0, jnp.float32)])

    w0, w1, w2, w3, g0, g1, g2, g3 = _stage1(ps2, pe2, dp2, ls, le, dl)

    ev_idx, ev_conf, cntv = _stage2(
        w0.reshape(NPAD), w1.reshape(NPAD), w2.reshape(NPAD), w3.reshape(NPAD),
        g0.reshape(NG), g1.reshape(NG), g2.reshape(NG), g3.reshape(NG),
        conf_p)

    ap = _stage3(conf_p.reshape(ROWS, 128), ev_conf, ev_idx, cntv)
    return ap[0, 0]


# speedup vs baseline: 2940.6514x; 1.1661x over previous
import functools

import jax
import jax.numpy as jnp
from jax import lax
from jax.experimental import pallas as pl
from jax.experimental.pallas import tpu as pltpu
from jax.experimental.pallas import tpu_sc as plsc

FPS = 25.0
N = 20000
M = 128
NPAD = 20480
ROWS = NPAD // 128
NG = NPAD // 16
NSG = NG // 16
BLK = 160
GRID1 = ROWS // BLK


def _iou_pack_kernel(ps_ref, pe_ref, dp_ref, ls_ref, le_ref, dl_ref,
                     w_ref, g_ref):
    ps = ps_ref[...]
    pe = pe_ref[...]
    dp = dp_ref[...]
    seg = (lax.broadcasted_iota(jnp.int32, (128, 8), 0) // 16 ==
           lax.broadcasted_iota(jnp.int32, (128, 8), 1)).astype(jnp.float32)
    accs = [jnp.zeros((BLK, 128), jnp.int32) for _ in range(4)]
    gaccs = [jnp.zeros((BLK, 8), jnp.int32) for _ in range(4)]
    for j in range(M):
        ls = ls_ref[j]
        le = le_ref[j]
        dl = dl_ref[j]
        imin = jnp.maximum(ps, ls)
        imax = jnp.minimum(pe, le)
        inter = jnp.maximum(imax - imin, jnp.float32(0.0))
        union = dp + dl - inter
        tpb = (inter / union) > jnp.float32(0.5)
        tp = tpb.astype(jnp.int32)
        cnts = jnp.dot(tpb.astype(jnp.float32), seg,
                       preferred_element_type=jnp.float32)
        gtp = (cnts > jnp.float32(0.5)).astype(jnp.int32)
        w = j // 32
        b = j % 32
        accs[w] = accs[w] | (tp << b)
        gaccs[w] = gaccs[w] | (gtp << b)
    for w in range(4):
        w_ref[pl.ds(w * ROWS, ROWS), :] = accs[w]
        g_ref[pl.ds(w * ROWS, ROWS), :] = gaccs[w]


def _stage1(ps2, pe2, dp2, ls, le, dl):
    blk = pl.BlockSpec((BLK, 128), lambda r: (r, 0))
    smem = pl.BlockSpec(memory_space=pltpu.SMEM)
    return pl.pallas_call(
        _iou_pack_kernel,
        grid=(GRID1,),
        in_specs=[blk, blk, blk, smem, smem, smem],
        out_specs=[pl.BlockSpec((4 * ROWS, 128), lambda r: (0, 0)),
                   pl.BlockSpec((4 * ROWS, 8), lambda r: (0, 0))],
        out_shape=[jax.ShapeDtypeStruct((4 * ROWS, 128), jnp.int32),
                   jax.ShapeDtypeStruct((4 * ROWS, 8), jnp.int32)],
    )(ps2, pe2, dp2, ls, le, dl)


def _sc_greedy_body(w_h, g_h, conf_h,
                    ev_idx_h, ev_conf_h, cnt_h,
                    w_v, g_v_ref, conf_v,
                    ev_idx_v, ev_conf_v, cnt_v, sem):
    first = (lax.axis_index("c") == 0) & (lax.axis_index("s") == 0)

    @pl.when(first)
    def _():
        cw = pltpu.async_copy(w_h, w_v, sem)
        cg = pltpu.async_copy(g_h, g_v_ref, sem)
        cc = pltpu.async_copy(conf_h, conf_v, sem)
        cw.wait()
        cg.wait()
        cc.wait()
        for k in range(8):
            ev_idx_v[pl.ds(k * 16, 16)] = jnp.full((16,), 2**30, jnp.int32)
            ev_conf_v[pl.ds(k * 16, 16)] = jnp.full((16,), 2.0, jnp.float32)

        lanes = lax.iota(jnp.int32, 16)
        lane0mask = lanes == 0
        zv = jnp.zeros((16,), jnp.int32)

        def cond(st):
            sg, _, _, _, _, cnt = st
            return (sg < NSG) & (cnt < M)

        def body(st):
            sg, d0, d1, d2, d3, cnt = st
            q0 = g_v_ref[pl.ds(sg * 16, 16)] & ~d0
            q1 = g_v_ref[pl.ds(NG + sg * 16, 16)] & ~d1
            q2 = g_v_ref[pl.ds(2 * NG + sg * 16, 16)] & ~d2
            q3 = g_v_ref[pl.ds(3 * NG + sg * 16, 16)] & ~d3
            gav = (q0 | q1 | q2 | q3) != 0
            gselb = plsc.all_reduce_population_count(gav) > 0
            gsel_v = plsc.all_reduce_ffs(gav)
            g_v = jnp.clip(sg * 16 + gsel_v, jnp.int32(0), jnp.int32(NG - 1))

            ridx = g_v * 16 + lanes
            raw0 = plsc.load_gather(w_v, [ridx])
            raw1 = plsc.load_gather(w_v, [ridx + NPAD])
            raw2 = plsc.load_gather(w_v, [ridx + 2 * NPAD])
            raw3 = plsc.load_gather(w_v, [ridx + 3 * NPAD])
            avm = ((raw0 & ~d0) | (raw1 & ~d1) | (raw2 & ~d2) | (raw3 & ~d3)) != 0
            laneb = plsc.all_reduce_population_count(avm) > 0
            lane_v = plsc.all_reduce_ffs(avm)
            rv = jnp.clip(g_v * 16 + lane_v, jnp.int32(0), jnp.int32(NPAD - 1))

            x0 = plsc.load_gather(w_v, [rv]) & ~d0
            x1 = plsc.load_gather(w_v, [rv + NPAD]) & ~d1
            x2 = plsc.load_gather(w_v, [rv + 2 * NPAD]) & ~d2
            x3 = plsc.load_gather(w_v, [rv + 3 * NPAD]) & ~d3
            hitv = gselb & laneb
            stalev = gselb & ~laneb
            t0 = (x0 != 0) & hitv
            t1 = (x1 != 0) & ~(x0 != 0) & hitv
            t2 = (x2 != 0) & ~((x0 != 0) | (x1 != 0)) & hitv
            t3 = (x3 != 0) & ~((x0 != 0) | (x1 != 0) | (x2 != 0)) & hitv
            nd0 = jnp.where(t0, d0 | (x0 & (zv - x0)), d0)
            nd1 = jnp.where(t1, d1 | (x1 & (zv - x1)), d1)
            nd2 = jnp.where(t2, d2 | (x2 & (zv - x2)), d2)
            nd3 = jnp.where(t3, d3 | (x3 & (zv - x3)), d3)

            cv = plsc.load_gather(conf_v, [rv])
            ev_slot = jnp.full((16,), cnt, jnp.int32)
            emask = lane0mask & hitv
            plsc.store_scatter(ev_idx_v, [ev_slot], rv, mask=emask)
            plsc.store_scatter(ev_conf_v, [ev_slot], cv, mask=emask)

            plsc.store_scatter(w_v, [rv], zv, mask=emask)
            plsc.store_scatter(w_v, [rv + NPAD], zv, mask=emask)
            plsc.store_scatter(w_v, [rv + 2 * NPAD], zv, mask=emask)
            plsc.store_scatter(w_v, [rv + 3 * NPAD], zv, mask=emask)
            smask = lane0mask & stalev
            plsc.store_scatter(g_v_ref, [g_v], zv, mask=smask)
            plsc.store_scatter(g_v_ref, [g_v + NG], zv, mask=smask)
            plsc.store_scatter(g_v_ref, [g_v + 2 * NG], zv, mask=smask)
            plsc.store_scatter(g_v_ref, [g_v + 3 * NG], zv, mask=smask)

            comb_v = (jnp.where(gselb, jnp.int32(2), jnp.int32(0)) |
                      jnp.where(laneb, jnp.int32(1), jnp.int32(0)))
            comb = comb_v[0]
            ghit = comb >= 2
            hit_ev = comb == 3
            nsg = jnp.where(ghit, sg, sg + 1)
            ncnt = jnp.where(hit_ev, cnt + 1, cnt)
            return (nsg, nd0, nd1, nd2, nd3, ncnt)

        z = jnp.int32(0)
        st = lax.while_loop(cond, body, (z, zv, zv, zv, zv, z))
        cnt_v[...] = jnp.full((16,), st[5], jnp.int32)
        pltpu.sync_copy(ev_idx_v, ev_idx_h)
        pltpu.sync_copy(ev_conf_v, ev_conf_h)
        pltpu.sync_copy(cnt_v, cnt_h)


def _stage2(w, g, conf):
    mesh = plsc.VectorSubcoreMesh(core_axis_name="c", subcore_axis_name="s")
    return pl.kernel(
        _sc_greedy_body,
        out_type=[
            jax.ShapeDtypeStruct((M,), jnp.int32),
            jax.ShapeDtypeStruct((M,), jnp.float32),
            jax.ShapeDtypeStruct((16,), jnp.int32),
        ],
        mesh=mesh,
        compiler_params=pltpu.CompilerParams(needs_layout_passes=False),
        scratch_types=[
            pltpu.VMEM((4 * NPAD,), jnp.int32),
            pltpu.VMEM((4 * NG,), jnp.int32),
            pltpu.VMEM((NPAD,), jnp.float32),
            pltpu.VMEM((M,), jnp.int32),
            pltpu.VMEM((M,), jnp.float32),
            pltpu.VMEM((16,), jnp.int32),
            pltpu.SemaphoreType.DMA,
        ],
    )(w, g, conf)


def _rank_ap_kernel(conf_ref, evc_ref, evi_ref, cnt_ref, ap_ref):
    conf = conf_ref[...]
    idx = (lax.broadcasted_iota(jnp.int32, (ROWS, 128), 0) * 128 +
           lax.broadcasted_iota(jnp.int32, (ROWS, 128), 1))
    lane = lax.broadcasted_iota(jnp.int32, (1, 128), 1)
    rank_row = jnp.zeros((1, 128), jnp.float32)
    for e in range(M):
        c = evc_ref[e]
        ie = evi_ref[e]
        gt = jnp.sum((conf > c).astype(jnp.float32))
        eqb = jnp.sum(((conf == c) & (idx < ie)).astype(jnp.float32))
        re = jnp.float32(1.0) + gt + eqb
        rank_row = rank_row + jnp.where(lane == e, re, jnp.float32(0.0))

    cnt = cnt_ref[0]
    vcol = lane < cnt
    vrow = lax.broadcasted_iota(jnp.int32, (128, 1), 0) < cnt
    eye = (lax.broadcasted_iota(jnp.int32, (128, 128), 0) ==
           lax.broadcasted_iota(jnp.int32, (128, 128), 1)).astype(jnp.float32)
    rank_col = jnp.sum(eye * rank_row, axis=1, keepdims=True)
    less = ((rank_col < rank_row) & vrow).astype(jnp.float32)
    lvl = jnp.float32(1.0) + jnp.sum(less, axis=0, keepdims=True)
    v = lvl / rank_row
    jrow = (lax.broadcasted_iota(jnp.int32, (128, 1), 0) + 1).astype(jnp.float32)
    a = jnp.where((lvl >= jrow) & vcol, v, jnp.float32(0.0))
    mx = jnp.max(a, axis=1, keepdims=True)
    ap_ref[...] = jnp.sum(mx, keepdims=True) * jnp.float32(1.0 / M)


def _stage3(conf2, ev_conf, ev_idx, cntv):
    smem = pl.BlockSpec(memory_space=pltpu.SMEM)
    return pl.pallas_call(
        _rank_ap_kernel,
        in_specs=[pl.BlockSpec((ROWS, 128), lambda: (0, 0)),
                  smem, smem, smem],
        out_specs=pl.BlockSpec((1, 1), lambda: (0, 0)),
        out_shape=jax.ShapeDtypeStruct((1, 1), jnp.float32),
    )(conf2, ev_conf, ev_idx, cntv)


@jax.jit
def kernel(proposals, labels):
    conf = proposals[:, 0]
    ps = proposals[:, 1] / FPS
    pe = proposals[:, 2] / FPS
    dp = pe - ps
    ls = labels[:, 0]
    le = labels[:, 1]
    dl = le - ls

    pad = NPAD - N
    ps2 = jnp.concatenate([ps, jnp.full((pad,), 1e9, jnp.float32)]).reshape(ROWS, 128)
    pe2 = jnp.concatenate([pe, jnp.full((pad,), 1e9 + 1.0, jnp.float32)]).reshape(ROWS, 128)
    dp2 = jnp.concatenate([dp, jnp.full((pad,), 1.0, jnp.float32)]).reshape(ROWS, 128)
    conf_p = jnp.concatenate([conf, jnp.full((pad,), -1.0, jnp.float32)])

    w, g = _stage1(ps2, pe2, dp2, ls, le, dl)

    ev_idx, ev_conf, cntv = _stage2(
        w.reshape(4 * NPAD), g.reshape(4 * NG), conf_p)

    ap = _stage3(conf_p.reshape(ROWS, 128), ev_conf, ev_idx, cntv)
    return ap[0, 0]
